# baseline (speedup 1.0000x reference)
import functools

import jax
import jax.numpy as jnp
from jax import lax
from jax.experimental import pallas as pl
from jax.experimental.pallas import tpu as pltpu
from jax.experimental.pallas import tpu_sc as plsc

B, S, H = 1, 2048, 1024
F = 4096
E = 8
K = 2
T = B * S
G = T * K

BM = 256
BN = 256
P = G + E * BM
M_TILES = P // BM
N_TILES = F // BN
BT = 256

EPAD = 128

_NEG_INF = float("-inf")


def _router_body(x_ref, wr_ref, i1_ref, i2_ref, w1_ref, w2_ref):
    logits = jnp.dot(x_ref[...], wr_ref[...], preferred_element_type=jnp.float32)
    eidx = lax.broadcasted_iota(jnp.int32, (T, EPAD), 1)
    logits = jnp.where(eidx < E, logits, _NEG_INF)
    m1 = jnp.max(logits, axis=1, keepdims=True)
    i1 = jnp.min(jnp.where(logits == m1, eidx, EPAD), axis=1, keepdims=True)
    masked = jnp.where(eidx == i1, _NEG_INF, logits)
    m2 = jnp.max(masked, axis=1, keepdims=True)
    i2 = jnp.min(jnp.where(masked == m2, eidx, EPAD), axis=1, keepdims=True)
    e = jnp.exp(m2 - m1)
    d = 1.0 + e
    i1_ref[...] = i1
    i2_ref[...] = i2
    w1_ref[...] = 1.0 / d
    w2_ref[...] = e / d


def _router_call(x2d, wr_pad):
    return pl.pallas_call(
        _router_body,
        out_shape=(
            jax.ShapeDtypeStruct((T, 1), jnp.int32),
            jax.ShapeDtypeStruct((T, 1), jnp.int32),
            jax.ShapeDtypeStruct((T, 1), jnp.float32),
            jax.ShapeDtypeStruct((T, 1), jnp.float32),
        ),
    )(x2d, wr_pad)


def _dispatch_meta(i1, i2):
    e_flat = jnp.concatenate([i1[:, 0], i2[:, 0]])
    onehot = (e_flat[:, None] == jnp.arange(E, dtype=jnp.int32)[None, :])
    counts = jnp.sum(onehot.astype(jnp.int32), axis=0)
    padded = ((counts + BM - 1) // BM) * BM
    pend = jnp.cumsum(padded)
    pstart = pend - padded
    ustart = jnp.cumsum(counts) - counts
    order = jnp.argsort(e_flat, stable=True)
    sorted_e = e_flat[order]
    within = jnp.arange(G, dtype=jnp.int32) - ustart[sorted_e]
    row_sorted = pstart[sorted_e] + within
    pos = jnp.zeros((G,), jnp.int32).at[order].set(row_sorted)
    tok_row = jnp.zeros((P,), jnp.int32).at[row_sorted].set(
        (order % T).astype(jnp.int32))
    used = (jnp.sum(padded) // BM).astype(jnp.int32).reshape(1)
    g_map = jnp.minimum(
        jnp.searchsorted(pend, jnp.arange(M_TILES, dtype=jnp.int32) * BM,
                         side='right'),
        E - 1).astype(jnp.int32)
    return pos, tok_row, g_map, used


SC_CORES = 2
SC_SUBCORES = 16


@functools.lru_cache(maxsize=None)
def _make_sc_gather(n_rows, n_src):
    nw = SC_CORES * SC_SUBCORES
    rows_per_w = n_rows // nw
    ch = 32
    n_chunks = rows_per_w // ch
    mesh = plsc.VectorSubcoreMesh(core_axis_name="c", subcore_axis_name="s",
                                  num_cores=SC_CORES, num_subcores=SC_SUBCORES)

    @functools.partial(
        pl.kernel,
        mesh=mesh,
        out_type=jax.ShapeDtypeStruct((n_rows, H), jnp.float32),
        scratch_types=[
            pltpu.VMEM((ch,), jnp.int32),
            pltpu.VMEM((ch, H), jnp.float32),
            pltpu.SemaphoreType.DMA,
        ],
    )
    def gather_k(src_hbm, idx_hbm, out_hbm, idx_v, rows_v, sem):
        wid = lax.axis_index("s") * SC_CORES + lax.axis_index("c")
        base = wid * rows_per_w

        def body(i, carry):
            off = base + i * ch
            pltpu.sync_copy(idx_hbm.at[pl.ds(off, ch)], idx_v)
            pltpu.async_copy(src_hbm.at[idx_v], rows_v, sem).wait()
            pltpu.sync_copy(rows_v, out_hbm.at[pl.ds(off, ch)])
            return carry

        lax.fori_loop(0, n_chunks, body, 0)

    return gather_k


def _ffn_body(g_ref, u_ref, xs_ref, w1_ref, w3_ref, w2_ref, out_ref):
    n = pl.program_id(0)
    m = pl.program_id(1)

    @pl.when(m < u_ref[0])
    def _():
        rows = pl.ds(m * BM, BM)
        xm = xs_ref[rows, :].astype(jnp.bfloat16)
        h1 = jnp.dot(xm, w1_ref[0].astype(jnp.bfloat16),
                     preferred_element_type=jnp.float32)
        h3 = jnp.dot(xm, w3_ref[0].astype(jnp.bfloat16),
                     preferred_element_type=jnp.float32)
        act = (h1 * jax.nn.sigmoid(h1) * h3).astype(jnp.bfloat16)
        contrib = jnp.dot(act, w2_ref[0].astype(jnp.bfloat16),
                          preferred_element_type=jnp.float32)

        @pl.when(n == 0)
        def _():
            out_ref[rows, :] = contrib

        @pl.when(n > 0)
        def _():
            out_ref[rows, :] += contrib


def _ffn_call(g_map, used, xs, W1, W3, W2):
    grid_spec = pltpu.PrefetchScalarGridSpec(
        num_scalar_prefetch=2,
        grid=(N_TILES, M_TILES),
        in_specs=[
            pl.BlockSpec((P, H), lambda n, m, g, u: (0, 0)),
            pl.BlockSpec((1, H, BN), lambda n, m, g, u: (g[m], 0, n)),
            pl.BlockSpec((1, H, BN), lambda n, m, g, u: (g[m], 0, n)),
            pl.BlockSpec((1, BN, H), lambda n, m, g, u: (g[m], n, 0)),
        ],
        out_specs=pl.BlockSpec((P, H), lambda n, m, g, u: (0, 0)),
    )
    return pl.pallas_call(
        _ffn_body,
        grid_spec=grid_spec,
        out_shape=jax.ShapeDtypeStruct((P, H), jnp.float32),
        compiler_params=pltpu.CompilerParams(
            dimension_semantics=("arbitrary", "arbitrary")),
    )(g_map, used, xs, W1, W3, W2)


def _combine_body(s0_ref, s1_ref, w1_ref, w2_ref, f_ref, ss0_ref, ss1_ref):
    s0 = s0_ref[...]
    s1 = s1_ref[...]
    f_ref[...] = w1_ref[...] * s0 + w2_ref[...] * s1
    ss0_ref[...] = jnp.sum(s0 * s0, axis=1, keepdims=True)
    ss1_ref[...] = jnp.sum(s1 * s1, axis=1, keepdims=True)


def _combine_call(sel, w1, w2):
    return pl.pallas_call(
        _combine_body,
        grid=(T // BT,),
        in_specs=[
            pl.BlockSpec((BT, H), lambda t: (t, 0)),
            pl.BlockSpec((BT, H), lambda t: (t + T // BT, 0)),
            pl.BlockSpec((BT, 1), lambda t: (t, 0)),
            pl.BlockSpec((BT, 1), lambda t: (t, 0)),
        ],
        out_specs=[
            pl.BlockSpec((BT, H), lambda t: (t, 0)),
            pl.BlockSpec((BT, 1), lambda t: (t, 0)),
            pl.BlockSpec((BT, 1), lambda t: (t, 0)),
        ],
        out_shape=(
            jax.ShapeDtypeStruct((T, H), jnp.float32),
            jax.ShapeDtypeStruct((T, 1), jnp.float32),
            jax.ShapeDtypeStruct((T, 1), jnp.float32),
        ),
    )(sel, sel, w1, w2)


def kernel(x, Wr, W1, W2, W3):
    x2d = x.reshape(T, H)
    wr_pad = jnp.zeros((H, EPAD), jnp.float32).at[:, :E].set(Wr)
    i1, i2, w1, w2 = _router_call(x2d, wr_pad)
    pos, tok_row, g_map, used = _dispatch_meta(i1, i2)
    xs = _make_sc_gather(P, T)(x2d, tok_row)
    ys = _ffn_call(g_map, used, xs, W1, W3, W2)
    sel = _make_sc_gather(G, P)(ys, pos)
    final, ss0, ss1 = _combine_call(sel, w1, w2)
    routing_weights = jnp.concatenate([w1, w2], axis=1).reshape(B, S, K)
    expert_indices = jnp.concatenate([i1, i2], axis=1).reshape(B, S, K)
    metrics = jnp.sqrt(jnp.concatenate([ss0, ss1], axis=1)).reshape(B, S, K)
    return final.reshape(B, S, H), routing_weights, expert_indices, metrics


# speedup vs baseline: 1.4482x; 1.4482x over previous
import functools

import jax
import jax.numpy as jnp
from jax import lax
from jax.experimental import pallas as pl
from jax.experimental.pallas import tpu as pltpu
from jax.experimental.pallas import tpu_sc as plsc

B, S, H = 1, 2048, 1024
F = 4096
E = 8
K = 2
T = B * S
G = T * K

BM = 128
BN = 512
P = G + E * BM
M_TILES = P // BM
N_TILES = F // BN
BT = 256

EPAD = 128

_NEG_INF = float("-inf")


def _router_body(x_ref, wr_ref, i1_ref, i2_ref, w1_ref, w2_ref):
    logits = jnp.dot(x_ref[...], wr_ref[...], preferred_element_type=jnp.float32)
    eidx = lax.broadcasted_iota(jnp.int32, (T, EPAD), 1)
    logits = jnp.where(eidx < E, logits, _NEG_INF)
    m1 = jnp.max(logits, axis=1, keepdims=True)
    i1 = jnp.min(jnp.where(logits == m1, eidx, EPAD), axis=1, keepdims=True)
    masked = jnp.where(eidx == i1, _NEG_INF, logits)
    m2 = jnp.max(masked, axis=1, keepdims=True)
    i2 = jnp.min(jnp.where(masked == m2, eidx, EPAD), axis=1, keepdims=True)
    e = jnp.exp(m2 - m1)
    d = 1.0 + e
    i1_ref[...] = i1
    i2_ref[...] = i2
    w1_ref[...] = 1.0 / d
    w2_ref[...] = e / d


def _router_call(x2d, wr_pad):
    return pl.pallas_call(
        _router_body,
        out_shape=(
            jax.ShapeDtypeStruct((T, 1), jnp.int32),
            jax.ShapeDtypeStruct((T, 1), jnp.int32),
            jax.ShapeDtypeStruct((T, 1), jnp.float32),
            jax.ShapeDtypeStruct((T, 1), jnp.float32),
        ),
    )(x2d, wr_pad)


def _dispatch_meta(i1, i2):
    e_flat = jnp.concatenate([i1[:, 0], i2[:, 0]])
    onehot = (e_flat[:, None] == jnp.arange(E, dtype=jnp.int32)[None, :])
    counts = jnp.sum(onehot.astype(jnp.int32), axis=0)
    padded = ((counts + BM - 1) // BM) * BM
    pend = jnp.cumsum(padded)
    pstart = pend - padded
    ustart = jnp.cumsum(counts) - counts
    order = jnp.argsort(e_flat, stable=True)
    sorted_e = e_flat[order]
    within = jnp.arange(G, dtype=jnp.int32) - ustart[sorted_e]
    row_sorted = pstart[sorted_e] + within
    pos = jnp.zeros((G,), jnp.int32).at[order].set(row_sorted)
    tok_row = jnp.zeros((P,), jnp.int32).at[row_sorted].set(
        (order % T).astype(jnp.int32))
    tile_start = (pstart // BM).astype(jnp.int32)
    tile_cnt = (padded // BM).astype(jnp.int32)
    return pos, tok_row, tile_start, tile_cnt


SC_CORES = 2
SC_SUBCORES = 16


@functools.lru_cache(maxsize=None)
def _make_sc_gather(n_rows, n_src):
    nw = SC_CORES * SC_SUBCORES
    rows_per_w = n_rows // nw
    ch = 32
    n_chunks = rows_per_w // ch
    mesh = plsc.VectorSubcoreMesh(core_axis_name="c", subcore_axis_name="s",
                                  num_cores=SC_CORES, num_subcores=SC_SUBCORES)

    @functools.partial(
        pl.kernel,
        mesh=mesh,
        out_type=jax.ShapeDtypeStruct((n_rows, H), jnp.float32),
        scratch_types=[
            pltpu.VMEM((ch,), jnp.int32),
            pltpu.VMEM((ch, H), jnp.float32),
            pltpu.SemaphoreType.DMA,
        ],
    )
    def gather_k(src_hbm, idx_hbm, out_hbm, idx_v, rows_v, sem):
        wid = lax.axis_index("s") * SC_CORES + lax.axis_index("c")
        base = wid * rows_per_w

        def body(i, carry):
            off = base + i * ch
            pltpu.sync_copy(idx_hbm.at[pl.ds(off, ch)], idx_v)
            pltpu.async_copy(src_hbm.at[idx_v], rows_v, sem).wait()
            pltpu.sync_copy(rows_v, out_hbm.at[pl.ds(off, ch)])
            return carry

        lax.fori_loop(0, n_chunks, body, 0)

    return gather_k


def _ffn_body(ts_ref, tc_ref, xs_ref, w1_ref, w3_ref, w2_ref, out_ref):
    e = pl.program_id(0)
    n = pl.program_id(1)
    t0 = ts_ref[e]
    nt = tc_ref[e]
    w1 = w1_ref[0].astype(jnp.bfloat16)
    w3 = w3_ref[0].astype(jnp.bfloat16)
    w2 = w2_ref[0].astype(jnp.bfloat16)

    def tile(t, carry):
        rows = pl.ds((t0 + t) * BM, BM)
        xm = xs_ref[rows, :].astype(jnp.bfloat16)
        h1 = jnp.dot(xm, w1, preferred_element_type=jnp.float32)
        h3 = jnp.dot(xm, w3, preferred_element_type=jnp.float32)
        act = (h1 * jax.nn.sigmoid(h1) * h3).astype(jnp.bfloat16)
        contrib = jnp.dot(act, w2, preferred_element_type=jnp.float32)

        @pl.when(n == 0)
        def _():
            out_ref[rows, :] = contrib

        @pl.when(n > 0)
        def _():
            out_ref[rows, :] += contrib

        return carry

    lax.fori_loop(0, nt, tile, 0)


def _ffn_call(tile_start, tile_cnt, xs, W1, W3, W2):
    grid_spec = pltpu.PrefetchScalarGridSpec(
        num_scalar_prefetch=2,
        grid=(E, N_TILES),
        in_specs=[
            pl.BlockSpec((P, H), lambda e, n, ts, tc: (0, 0)),
            pl.BlockSpec((1, H, BN), lambda e, n, ts, tc: (e, 0, n)),
            pl.BlockSpec((1, H, BN), lambda e, n, ts, tc: (e, 0, n)),
            pl.BlockSpec((1, BN, H), lambda e, n, ts, tc: (e, n, 0)),
        ],
        out_specs=pl.BlockSpec((P, H), lambda e, n, ts, tc: (0, 0)),
    )
    return pl.pallas_call(
        _ffn_body,
        grid_spec=grid_spec,
        out_shape=jax.ShapeDtypeStruct((P, H), jnp.float32),
        compiler_params=pltpu.CompilerParams(
            dimension_semantics=("arbitrary", "arbitrary")),
    )(tile_start, tile_cnt, xs, W1, W3, W2)


def _combine_body(s0_ref, s1_ref, w1_ref, w2_ref, f_ref, ss0_ref, ss1_ref):
    s0 = s0_ref[...]
    s1 = s1_ref[...]
    f_ref[...] = w1_ref[...] * s0 + w2_ref[...] * s1
    ss0_ref[...] = jnp.sum(s0 * s0, axis=1, keepdims=True)
    ss1_ref[...] = jnp.sum(s1 * s1, axis=1, keepdims=True)


def _combine_call(sel, w1, w2):
    return pl.pallas_call(
        _combine_body,
        grid=(T // BT,),
        in_specs=[
            pl.BlockSpec((BT, H), lambda t: (t, 0)),
            pl.BlockSpec((BT, H), lambda t: (t + T // BT, 0)),
            pl.BlockSpec((BT, 1), lambda t: (t, 0)),
            pl.BlockSpec((BT, 1), lambda t: (t, 0)),
        ],
        out_specs=[
            pl.BlockSpec((BT, H), lambda t: (t, 0)),
            pl.BlockSpec((BT, 1), lambda t: (t, 0)),
            pl.BlockSpec((BT, 1), lambda t: (t, 0)),
        ],
        out_shape=(
            jax.ShapeDtypeStruct((T, H), jnp.float32),
            jax.ShapeDtypeStruct((T, 1), jnp.float32),
            jax.ShapeDtypeStruct((T, 1), jnp.float32),
        ),
    )(sel, sel, w1, w2)


def kernel(x, Wr, W1, W2, W3):
    x2d = x.reshape(T, H)
    wr_pad = jnp.zeros((H, EPAD), jnp.float32).at[:, :E].set(Wr)
    i1, i2, w1, w2 = _router_call(x2d, wr_pad)
    pos, tok_row, tile_start, tile_cnt = _dispatch_meta(i1, i2)
    xs = _make_sc_gather(P, T)(x2d, tok_row)
    ys = _ffn_call(tile_start, tile_cnt, xs, W1, W3, W2)
    sel = _make_sc_gather(G, P)(ys, pos)
    final, ss0, ss1 = _combine_call(sel, w1, w2)
    routing_weights = jnp.concatenate([w1, w2], axis=1).reshape(B, S, K)
    expert_indices = jnp.concatenate([i1, i2], axis=1).reshape(B, S, K)
    metrics = jnp.sqrt(jnp.concatenate([ss0, ss1], axis=1)).reshape(B, S, K)
    return final.reshape(B, S, H), routing_weights, expert_indices, metrics


# speedup vs baseline: 1.4742x; 1.0180x over previous
import functools

import jax
import jax.numpy as jnp
from jax import lax
from jax.experimental import pallas as pl
from jax.experimental.pallas import tpu as pltpu
from jax.experimental.pallas import tpu_sc as plsc

B, S, H = 1, 2048, 1024
F = 4096
E = 8
K = 2
T = B * S
G = T * K

BM = 128
BN = 512
P = G + E * BM
M_TILES = P // BM
N_TILES = F // BN
BT = 256

EPAD = 128

_NEG_INF = float("-inf")


def _router_body(x_ref, wr_ref, i1_ref, i2_ref, w1_ref, w2_ref):
    logits = jnp.dot(x_ref[...], wr_ref[...], preferred_element_type=jnp.float32)
    eidx = lax.broadcasted_iota(jnp.int32, (T, EPAD), 1)
    logits = jnp.where(eidx < E, logits, _NEG_INF)
    m1 = jnp.max(logits, axis=1, keepdims=True)
    i1 = jnp.min(jnp.where(logits == m1, eidx, EPAD), axis=1, keepdims=True)
    masked = jnp.where(eidx == i1, _NEG_INF, logits)
    m2 = jnp.max(masked, axis=1, keepdims=True)
    i2 = jnp.min(jnp.where(masked == m2, eidx, EPAD), axis=1, keepdims=True)
    e = jnp.exp(m2 - m1)
    d = 1.0 + e
    i1_ref[...] = i1
    i2_ref[...] = i2
    w1_ref[...] = 1.0 / d
    w2_ref[...] = e / d


def _router_call(x2d, wr_pad):
    return pl.pallas_call(
        _router_body,
        out_shape=(
            jax.ShapeDtypeStruct((T, 1), jnp.int32),
            jax.ShapeDtypeStruct((T, 1), jnp.int32),
            jax.ShapeDtypeStruct((T, 1), jnp.float32),
            jax.ShapeDtypeStruct((T, 1), jnp.float32),
        ),
    )(x2d, wr_pad)


def _dispatch_meta(i1, i2):
    e_flat = jnp.concatenate([i1[:, 0], i2[:, 0]])
    onehot = (e_flat[:, None] == jnp.arange(E, dtype=jnp.int32)[None, :])
    counts = jnp.sum(onehot.astype(jnp.int32), axis=0)
    padded = ((counts + BM - 1) // BM) * BM
    pend = jnp.cumsum(padded)
    pstart = pend - padded
    ustart = jnp.cumsum(counts) - counts
    order = jnp.argsort(e_flat, stable=True)
    sorted_e = e_flat[order]
    within = jnp.arange(G, dtype=jnp.int32) - ustart[sorted_e]
    row_sorted = pstart[sorted_e] + within
    pos = jnp.zeros((G,), jnp.int32).at[order].set(row_sorted)
    tok_row = jnp.zeros((P,), jnp.int32).at[row_sorted].set(
        (order % T).astype(jnp.int32))
    tile_start = (pstart // BM).astype(jnp.int32)
    tile_cnt = (padded // BM).astype(jnp.int32)
    return pos, tok_row, tile_start, tile_cnt


SC_CORES = 2
SC_SUBCORES = 16


@functools.lru_cache(maxsize=None)
def _make_sc_gather(n_rows, n_src):
    nw = SC_CORES * SC_SUBCORES
    rows_per_w = n_rows // nw
    ch = 32
    n_chunks = rows_per_w // ch
    mesh = plsc.VectorSubcoreMesh(core_axis_name="c", subcore_axis_name="s",
                                  num_cores=SC_CORES, num_subcores=SC_SUBCORES)

    @functools.partial(
        pl.kernel,
        mesh=mesh,
        out_type=jax.ShapeDtypeStruct((n_rows, H), jnp.float32),
        scratch_types=[
            pltpu.VMEM((n_chunks, ch), jnp.int32),
            pltpu.VMEM((ch, H), jnp.float32),
            pltpu.VMEM((ch, H), jnp.float32),
            pltpu.SemaphoreType.DMA,
            pltpu.SemaphoreType.DMA,
            pltpu.SemaphoreType.DMA,
            pltpu.SemaphoreType.DMA,
        ],
    )
    def gather_k(src_hbm, idx_hbm, out_hbm, idx_v, buf0, buf1, g0, g1, w0, w1):
        wid = lax.axis_index("s") * SC_CORES + lax.axis_index("c")
        base = wid * rows_per_w
        pltpu.sync_copy(idx_hbm.at[wid], idx_v)
        bufs = (buf0, buf1)
        gsems = (g0, g1)
        wsems = (w0, w1)

        def out_at(i):
            return out_hbm.at[pl.ds(base + i * ch, ch)]

        pltpu.async_copy(src_hbm.at[idx_v.at[0]], buf0, g0)
        if n_chunks > 1:
            pltpu.async_copy(src_hbm.at[idx_v.at[1]], buf1, g1)
        for i in range(n_chunks):
            b = i % 2
            pltpu.make_async_copy(src_hbm.at[idx_v.at[i]], bufs[b],
                                  gsems[b]).wait()
            pltpu.async_copy(bufs[b], out_at(i), wsems[b])
            if i + 2 < n_chunks:
                pltpu.make_async_copy(bufs[b], out_at(i), wsems[b]).wait()
                pltpu.async_copy(src_hbm.at[idx_v.at[i + 2]], bufs[b],
                                 gsems[b])
        for i in range(max(0, n_chunks - 2), n_chunks):
            b = i % 2
            pltpu.make_async_copy(bufs[b], out_at(i), wsems[b]).wait()

    def call(src, idx):
        return gather_k(src, idx.reshape(nw, n_chunks, ch))

    return call


def _ffn_body(ts_ref, tc_ref, xs_ref, w1_ref, w3_ref, w2_ref, out_ref):
    e = pl.program_id(0)
    n = pl.program_id(1)
    t0 = ts_ref[e]
    nt = tc_ref[e]
    w1 = w1_ref[0].astype(jnp.bfloat16)
    w3 = w3_ref[0].astype(jnp.bfloat16)
    w2 = w2_ref[0].astype(jnp.bfloat16)

    def tile(t, carry):
        rows = pl.ds((t0 + t) * BM, BM)
        xm = xs_ref[rows, :].astype(jnp.bfloat16)
        h1 = jnp.dot(xm, w1, preferred_element_type=jnp.float32)
        h3 = jnp.dot(xm, w3, preferred_element_type=jnp.float32)
        act = (h1 * jax.nn.sigmoid(h1) * h3).astype(jnp.bfloat16)
        contrib = jnp.dot(act, w2, preferred_element_type=jnp.float32)

        @pl.when(n == 0)
        def _():
            out_ref[rows, :] = contrib

        @pl.when(n > 0)
        def _():
            out_ref[rows, :] += contrib

        return carry

    lax.fori_loop(0, nt, tile, 0)


def _ffn_call(tile_start, tile_cnt, xs, W1, W3, W2):
    grid_spec = pltpu.PrefetchScalarGridSpec(
        num_scalar_prefetch=2,
        grid=(E, N_TILES),
        in_specs=[
            pl.BlockSpec((P, H), lambda e, n, ts, tc: (0, 0)),
            pl.BlockSpec((1, H, BN), lambda e, n, ts, tc: (e, 0, n)),
            pl.BlockSpec((1, H, BN), lambda e, n, ts, tc: (e, 0, n)),
            pl.BlockSpec((1, BN, H), lambda e, n, ts, tc: (e, n, 0)),
        ],
        out_specs=pl.BlockSpec((P, H), lambda e, n, ts, tc: (0, 0)),
    )
    return pl.pallas_call(
        _ffn_body,
        grid_spec=grid_spec,
        out_shape=jax.ShapeDtypeStruct((P, H), jnp.float32),
        compiler_params=pltpu.CompilerParams(
            dimension_semantics=("arbitrary", "arbitrary")),
    )(tile_start, tile_cnt, xs, W1, W3, W2)


def _combine_body(s0_ref, s1_ref, w1_ref, w2_ref, f_ref, ss0_ref, ss1_ref):
    s0 = s0_ref[...]
    s1 = s1_ref[...]
    f_ref[...] = w1_ref[...] * s0 + w2_ref[...] * s1
    ss0_ref[...] = jnp.sum(s0 * s0, axis=1, keepdims=True)
    ss1_ref[...] = jnp.sum(s1 * s1, axis=1, keepdims=True)


def _combine_call(sel, w1, w2):
    return pl.pallas_call(
        _combine_body,
        grid=(T // BT,),
        in_specs=[
            pl.BlockSpec((BT, H), lambda t: (t, 0)),
            pl.BlockSpec((BT, H), lambda t: (t + T // BT, 0)),
            pl.BlockSpec((BT, 1), lambda t: (t, 0)),
            pl.BlockSpec((BT, 1), lambda t: (t, 0)),
        ],
        out_specs=[
            pl.BlockSpec((BT, H), lambda t: (t, 0)),
            pl.BlockSpec((BT, 1), lambda t: (t, 0)),
            pl.BlockSpec((BT, 1), lambda t: (t, 0)),
        ],
        out_shape=(
            jax.ShapeDtypeStruct((T, H), jnp.float32),
            jax.ShapeDtypeStruct((T, 1), jnp.float32),
            jax.ShapeDtypeStruct((T, 1), jnp.float32),
        ),
    )(sel, sel, w1, w2)


def kernel(x, Wr, W1, W2, W3):
    x2d = x.reshape(T, H)
    wr_pad = jnp.zeros((H, EPAD), jnp.float32).at[:, :E].set(Wr)
    i1, i2, w1, w2 = _router_call(x2d, wr_pad)
    pos, tok_row, tile_start, tile_cnt = _dispatch_meta(i1, i2)
    xs = _make_sc_gather(P, T)(x2d, tok_row)
    ys = _ffn_call(tile_start, tile_cnt, xs, W1, W3, W2)
    sel = _make_sc_gather(G, P)(ys, pos)
    final, ss0, ss1 = _combine_call(sel, w1, w2)
    routing_weights = jnp.concatenate([w1, w2], axis=1).reshape(B, S, K)
    expert_indices = jnp.concatenate([i1, i2], axis=1).reshape(B, S, K)
    metrics = jnp.sqrt(jnp.concatenate([ss0, ss1], axis=1)).reshape(B, S, K)
    return final.reshape(B, S, H), routing_weights, expert_indices, metrics


# speedup vs baseline: 1.4767x; 1.0017x over previous
import functools

import jax
import jax.numpy as jnp
from jax import lax
from jax.experimental import pallas as pl
from jax.experimental.pallas import tpu as pltpu
from jax.experimental.pallas import tpu_sc as plsc

B, S, H = 1, 2048, 1024
F = 4096
E = 8
K = 2
T = B * S
G = T * K

BM = 128
BN = 512
P = G + E * BM
M_TILES = P // BM
N_TILES = F // BN
BT = 256

EPAD = 128

_NEG_INF = float("-inf")


def _router_body(x_ref, wr_ref, i1_ref, i2_ref, w1_ref, w2_ref):
    logits = jnp.dot(x_ref[...], wr_ref[...], preferred_element_type=jnp.float32)
    eidx = lax.broadcasted_iota(jnp.int32, (T, EPAD), 1)
    logits = jnp.where(eidx < E, logits, _NEG_INF)
    m1 = jnp.max(logits, axis=1, keepdims=True)
    i1 = jnp.min(jnp.where(logits == m1, eidx, EPAD), axis=1, keepdims=True)
    masked = jnp.where(eidx == i1, _NEG_INF, logits)
    m2 = jnp.max(masked, axis=1, keepdims=True)
    i2 = jnp.min(jnp.where(masked == m2, eidx, EPAD), axis=1, keepdims=True)
    e = jnp.exp(m2 - m1)
    d = 1.0 + e
    i1_ref[...] = i1
    i2_ref[...] = i2
    w1_ref[...] = 1.0 / d
    w2_ref[...] = e / d


def _router_call(x2d, wr_pad):
    return pl.pallas_call(
        _router_body,
        out_shape=(
            jax.ShapeDtypeStruct((T, 1), jnp.int32),
            jax.ShapeDtypeStruct((T, 1), jnp.int32),
            jax.ShapeDtypeStruct((T, 1), jnp.float32),
            jax.ShapeDtypeStruct((T, 1), jnp.float32),
        ),
    )(x2d, wr_pad)


def _dispatch_meta(i1, i2):
    e_flat = jnp.concatenate([i1[:, 0], i2[:, 0]])
    onehot = (e_flat[:, None] ==
              jnp.arange(E, dtype=jnp.int32)[None, :]).astype(jnp.int32)
    csum = jnp.cumsum(onehot, axis=0)
    counts = csum[-1]
    padded = ((counts + BM - 1) // BM) * BM
    pstart = jnp.cumsum(padded) - padded
    rank = jnp.take_along_axis(csum, e_flat[:, None], axis=1)[:, 0] - 1
    pos = pstart[e_flat] + rank
    tok_row = jnp.zeros((P,), jnp.int32).at[pos].set(
        (jnp.arange(G, dtype=jnp.int32) % T))
    tile_start = (pstart // BM).astype(jnp.int32)
    tile_cnt = (padded // BM).astype(jnp.int32)
    return pos, tok_row, tile_start, tile_cnt


SC_CORES = 2
SC_SUBCORES = 16


@functools.lru_cache(maxsize=None)
def _make_sc_gather(n_rows, n_src):
    nw = SC_CORES * SC_SUBCORES
    rows_per_w = n_rows // nw
    ch = 32
    n_chunks = rows_per_w // ch
    mesh = plsc.VectorSubcoreMesh(core_axis_name="c", subcore_axis_name="s",
                                  num_cores=SC_CORES, num_subcores=SC_SUBCORES)

    @functools.partial(
        pl.kernel,
        mesh=mesh,
        out_type=jax.ShapeDtypeStruct((n_rows, H), jnp.float32),
        scratch_types=[
            pltpu.VMEM((n_chunks, ch), jnp.int32),
            pltpu.VMEM((ch, H), jnp.float32),
            pltpu.VMEM((ch, H), jnp.float32),
            pltpu.SemaphoreType.DMA,
            pltpu.SemaphoreType.DMA,
            pltpu.SemaphoreType.DMA,
            pltpu.SemaphoreType.DMA,
        ],
    )
    def gather_k(src_hbm, idx_hbm, out_hbm, idx_v, buf0, buf1, g0, g1, w0, w1):
        wid = lax.axis_index("s") * SC_CORES + lax.axis_index("c")
        base = wid * rows_per_w
        pltpu.sync_copy(idx_hbm.at[wid], idx_v)
        bufs = (buf0, buf1)
        gsems = (g0, g1)
        wsems = (w0, w1)

        def out_at(i):
            return out_hbm.at[pl.ds(base + i * ch, ch)]

        pltpu.async_copy(src_hbm.at[idx_v.at[0]], buf0, g0)
        if n_chunks > 1:
            pltpu.async_copy(src_hbm.at[idx_v.at[1]], buf1, g1)
        for i in range(n_chunks):
            b = i % 2
            pltpu.make_async_copy(src_hbm.at[idx_v.at[i]], bufs[b],
                                  gsems[b]).wait()
            pltpu.async_copy(bufs[b], out_at(i), wsems[b])
            if i + 2 < n_chunks:
                pltpu.make_async_copy(bufs[b], out_at(i), wsems[b]).wait()
                pltpu.async_copy(src_hbm.at[idx_v.at[i + 2]], bufs[b],
                                 gsems[b])
        for i in range(max(0, n_chunks - 2), n_chunks):
            b = i % 2
            pltpu.make_async_copy(bufs[b], out_at(i), wsems[b]).wait()

    def call(src, idx):
        return gather_k(src, idx.reshape(nw, n_chunks, ch))

    return call


def _ffn_body(ts_ref, tc_ref, xs_ref, w1_ref, w3_ref, w2_ref, out_ref):
    e = pl.program_id(0)
    n = pl.program_id(1)
    t0 = ts_ref[e]
    nt = tc_ref[e]
    w1 = w1_ref[0].astype(jnp.bfloat16)
    w3 = w3_ref[0].astype(jnp.bfloat16)
    w2 = w2_ref[0].astype(jnp.bfloat16)

    def tile(t, carry):
        rows = pl.ds((t0 + t) * BM, BM)
        xm = xs_ref[rows, :].astype(jnp.bfloat16)
        h1 = jnp.dot(xm, w1, preferred_element_type=jnp.float32)
        h3 = jnp.dot(xm, w3, preferred_element_type=jnp.float32)
        act = (h1 * jax.nn.sigmoid(h1) * h3).astype(jnp.bfloat16)
        contrib = jnp.dot(act, w2, preferred_element_type=jnp.float32)

        @pl.when(n == 0)
        def _():
            out_ref[rows, :] = contrib

        @pl.when(n > 0)
        def _():
            out_ref[rows, :] += contrib

        return carry

    lax.fori_loop(0, nt, tile, 0)


def _ffn_call(tile_start, tile_cnt, xs, W1, W3, W2):
    grid_spec = pltpu.PrefetchScalarGridSpec(
        num_scalar_prefetch=2,
        grid=(E, N_TILES),
        in_specs=[
            pl.BlockSpec((P, H), lambda e, n, ts, tc: (0, 0)),
            pl.BlockSpec((1, H, BN), lambda e, n, ts, tc: (e, 0, n)),
            pl.BlockSpec((1, H, BN), lambda e, n, ts, tc: (e, 0, n)),
            pl.BlockSpec((1, BN, H), lambda e, n, ts, tc: (e, n, 0)),
        ],
        out_specs=pl.BlockSpec((P, H), lambda e, n, ts, tc: (0, 0)),
    )
    return pl.pallas_call(
        _ffn_body,
        grid_spec=grid_spec,
        out_shape=jax.ShapeDtypeStruct((P, H), jnp.float32),
        compiler_params=pltpu.CompilerParams(
            dimension_semantics=("arbitrary", "arbitrary")),
    )(tile_start, tile_cnt, xs, W1, W3, W2)


def _combine_body(s0_ref, s1_ref, w1_ref, w2_ref, f_ref, ss0_ref, ss1_ref):
    s0 = s0_ref[...]
    s1 = s1_ref[...]
    f_ref[...] = w1_ref[...] * s0 + w2_ref[...] * s1
    ss0_ref[...] = jnp.sum(s0 * s0, axis=1, keepdims=True)
    ss1_ref[...] = jnp.sum(s1 * s1, axis=1, keepdims=True)


def _combine_call(sel, w1, w2):
    return pl.pallas_call(
        _combine_body,
        grid=(T // BT,),
        in_specs=[
            pl.BlockSpec((BT, H), lambda t: (t, 0)),
            pl.BlockSpec((BT, H), lambda t: (t + T // BT, 0)),
            pl.BlockSpec((BT, 1), lambda t: (t, 0)),
            pl.BlockSpec((BT, 1), lambda t: (t, 0)),
        ],
        out_specs=[
            pl.BlockSpec((BT, H), lambda t: (t, 0)),
            pl.BlockSpec((BT, 1), lambda t: (t, 0)),
            pl.BlockSpec((BT, 1), lambda t: (t, 0)),
        ],
        out_shape=(
            jax.ShapeDtypeStruct((T, H), jnp.float32),
            jax.ShapeDtypeStruct((T, 1), jnp.float32),
            jax.ShapeDtypeStruct((T, 1), jnp.float32),
        ),
    )(sel, sel, w1, w2)


def kernel(x, Wr, W1, W2, W3):
    x2d = x.reshape(T, H)
    wr_pad = jnp.zeros((H, EPAD), jnp.float32).at[:, :E].set(Wr)
    i1, i2, w1, w2 = _router_call(x2d, wr_pad)
    pos, tok_row, tile_start, tile_cnt = _dispatch_meta(i1, i2)
    xs = _make_sc_gather(P, T)(x2d, tok_row)
    ys = _ffn_call(tile_start, tile_cnt, xs, W1, W3, W2)
    sel = _make_sc_gather(G, P)(ys, pos)
    final, ss0, ss1 = _combine_call(sel, w1, w2)
    routing_weights = jnp.concatenate([w1, w2], axis=1).reshape(B, S, K)
    expert_indices = jnp.concatenate([i1, i2], axis=1).reshape(B, S, K)
    metrics = jnp.sqrt(jnp.concatenate([ss0, ss1], axis=1)).reshape(B, S, K)
    return final.reshape(B, S, H), routing_weights, expert_indices, metrics
